# baseline (speedup 1.0000x reference)
import functools

import jax
import jax.numpy as jnp
from jax import lax
from jax.experimental import pallas as pl
from jax.experimental.pallas import tpu as pltpu, tpu_sc as plsc

_CHUNK = 128


@functools.lru_cache(maxsize=None)
def _make_gather(V, D, B):
    info = plsc.get_sparse_core_info()
    NC, NS = info.num_cores, info.num_subcores
    NW = NC * NS
    assert B % (NW * _CHUNK) == 0
    n_chunks = B // (NW * _CHUNK)
    mesh = plsc.VectorSubcoreMesh(core_axis_name="c", subcore_axis_name="s")

    @functools.partial(
        pl.kernel,
        mesh=mesh,
        compiler_params=pltpu.CompilerParams(use_tc_tiling_on_sc=False),
        out_type=jax.ShapeDtypeStruct((NW, n_chunks, _CHUNK, D), jnp.float32),
        scratch_types=[
            pltpu.VMEM((n_chunks, _CHUNK), jnp.int32),
            pltpu.VMEM((n_chunks, _CHUNK, D), jnp.float32),
            pltpu.SemaphoreType.DMA,
        ],
    )
    def gather_kernel(idx_hbm, table_hbm, out_hbm, idx_v, rows_v, sem):
        wid = lax.axis_index("s") * NC + lax.axis_index("c")
        pltpu.sync_copy(idx_hbm.at[wid], idx_v)
        copies = [
            pltpu.async_copy(table_hbm.at[idx_v.at[j]], rows_v.at[j], sem)
            for j in range(n_chunks)
        ]
        for c in copies:
            c.wait()
        pltpu.sync_copy(rows_v, out_hbm.at[wid])

    return gather_kernel


def kernel(indices, embedding_weight):
    V, D = embedding_weight.shape
    (B,) = indices.shape
    info = plsc.get_sparse_core_info()
    NW = info.num_cores * info.num_subcores
    idx = indices.astype(jnp.int32).reshape(NW, B // (NW * _CHUNK), _CHUNK)
    out = _make_gather(V, D, B)(idx, embedding_weight)
    return out.reshape(B, D)


# speedup vs baseline: 2.3143x; 2.3143x over previous
import functools

import jax
import jax.numpy as jnp
from jax import lax
from jax.experimental import pallas as pl
from jax.experimental.pallas import tpu as pltpu, tpu_sc as plsc

_G = 16


@functools.lru_cache(maxsize=None)
def _make_gather(V, D, B):
    info = plsc.get_sparse_core_info()
    NC, NS = info.num_cores, info.num_subcores
    NW = NC * NS
    BPW = B // NW
    NCH = BPW // _G
    assert B % (NW * _G) == 0 and NCH % 2 == 0 and D == 64 and V % 8 == 0
    mesh = plsc.VectorSubcoreMesh(core_axis_name="c", subcore_axis_name="s")

    @functools.partial(
        pl.kernel,
        mesh=mesh,
        out_type=jax.ShapeDtypeStruct((B, D), jnp.float32),
        scratch_types=[
            pltpu.VMEM((BPW,), jnp.int32),
            pltpu.VMEM((2, _G, 8, D), jnp.float32),
            pltpu.VMEM((BPW, D), jnp.float32),
            pltpu.SemaphoreType.DMA,
            pltpu.SemaphoreType.DMA,
        ],
    )
    def gather_kernel(idx_hbm, tbl_hbm, out_hbm, idx_v, tiles_v, sel_v, sem0, sem1):
        wid = lax.axis_index("s") * NC + lax.axis_index("c")
        base = wid * BPW
        pltpu.sync_copy(idx_hbm.at[pl.ds(base, BPW)], idx_v)
        sems = [sem0, sem1]

        def fire(ch, buf):
            vvec = idx_v[pl.ds(pl.multiple_of(ch * _G, _G), _G)]
            for j in range(_G):
                v8 = lax.shift_right_logical(vvec[j], 3)
                pltpu.async_copy(tbl_hbm.at[v8], tiles_v.at[buf, j], sems[buf])

        def drain(buf):
            for j in range(_G):
                pltpu.make_async_copy(
                    tbl_hbm.at[0], tiles_v.at[buf, j], sems[buf]
                ).wait()

        def select(ch, buf):
            o = pl.multiple_of(ch * _G, _G)
            vvec = idx_v[pl.ds(o, _G)]
            for j in range(_G):
                par = lax.bitwise_and(vvec[j], 7)
                for g in range(D // 16):
                    sel_v[o + j, pl.ds(g * 16, 16)] = (
                        tiles_v[buf, j, par, pl.ds(g * 16, 16)]
                    )

        fire(0, 0)

        @pl.loop(0, NCH // 2)
        def _(t):
            ch0 = t * 2
            fire(ch0 + 1, 1)
            drain(0)
            select(ch0, 0)

            @pl.when(ch0 + 2 < NCH)
            def _():
                fire(ch0 + 2, 0)

            drain(1)
            select(ch0 + 1, 1)

        pltpu.sync_copy(sel_v, out_hbm.at[pl.ds(base, BPW)])

    return gather_kernel


def kernel(indices, embedding_weight):
    V, D = embedding_weight.shape
    (B,) = indices.shape
    tbl = embedding_weight.reshape(V // 8, 8, D)
    return _make_gather(V, D, B)(indices.astype(jnp.int32), tbl)


# speedup vs baseline: 2.3513x; 1.0160x over previous
import functools

import jax
import jax.numpy as jnp
from jax import lax
from jax.experimental import pallas as pl
from jax.experimental.pallas import tpu as pltpu, tpu_sc as plsc

_G = 16
_NBUF = 4


@functools.lru_cache(maxsize=None)
def _make_gather(V, D, B):
    info = plsc.get_sparse_core_info()
    NC, NS = info.num_cores, info.num_subcores
    NW = NC * NS
    BPW = B // NW
    NCH = BPW // _G
    assert B % (NW * _G) == 0 and NCH % _NBUF == 0 and D == 64 and V % 8 == 0
    mesh = plsc.VectorSubcoreMesh(core_axis_name="c", subcore_axis_name="s")

    @functools.partial(
        pl.kernel,
        mesh=mesh,
        out_type=jax.ShapeDtypeStruct((B, D), jnp.float32),
        scratch_types=[
            pltpu.VMEM((BPW,), jnp.int32),
            pltpu.VMEM((_NBUF, _G, 8, D), jnp.float32),
            pltpu.VMEM((_NBUF, _G, D), jnp.float32),
        ] + [pltpu.SemaphoreType.DMA] * 8,
    )
    def gather_kernel(idx_hbm, tbl_hbm, out_hbm, idx_v, tiles_v, osel_v, *sems8):
        wid = lax.axis_index("s") * NC + lax.axis_index("c")
        base = wid * BPW
        pltpu.sync_copy(idx_hbm.at[pl.ds(base, BPW)], idx_v)
        sems, osems = list(sems8[:4]), list(sems8[4:])

        def fire(ch, buf):
            vvec = idx_v[pl.ds(pl.multiple_of(ch * _G, _G), _G)]
            for j in range(_G):
                v8 = lax.shift_right_logical(vvec[j], 3)
                pltpu.async_copy(tbl_hbm.at[v8], tiles_v.at[buf, j], sems[buf])

        def drain(buf):
            for j in range(_G):
                pltpu.make_async_copy(
                    tbl_hbm.at[0], tiles_v.at[buf, j], sems[buf]
                ).wait()

        def owait(buf):
            pltpu.make_async_copy(
                osel_v.at[buf], out_hbm.at[pl.ds(0, _G)], osems[buf]
            ).wait()

        def select_and_out(ch, buf):
            vvec = idx_v[pl.ds(pl.multiple_of(ch * _G, _G), _G)]
            for j in range(_G):
                par = lax.bitwise_and(vvec[j], 7)
                for g in range(D // 16):
                    osel_v[buf, j, pl.ds(g * 16, 16)] = (
                        tiles_v[buf, j, par, pl.ds(g * 16, 16)]
                    )
            pltpu.async_copy(
                osel_v.at[buf], out_hbm.at[pl.ds(base + ch * _G, _G)], osems[buf]
            )

        for b in range(_NBUF - 1):
            fire(b, b)

        @pl.loop(0, NCH // _NBUF)
        def _(t):
            ch0 = t * _NBUF
            for k in range(_NBUF):
                ch = ch0 + k
                drain(k)

                @pl.when(ch >= _NBUF)
                def _():
                    owait(k)

                select_and_out(ch, k)

                @pl.when(ch + _NBUF - 1 < NCH)
                def _():
                    fire(ch + _NBUF - 1, (k + _NBUF - 1) % _NBUF)

        for b in range(_NBUF):
            owait(b)

    return gather_kernel


def kernel(indices, embedding_weight):
    V, D = embedding_weight.shape
    (B,) = indices.shape
    tbl = embedding_weight.reshape(V // 8, 8, D)
    return _make_gather(V, D, B)(indices.astype(jnp.int32), tbl)
